# Initial kernel scaffold; baseline (speedup 1.0000x reference)
import jax
import jax.numpy as jnp
from jax.experimental import pallas as pl


def kernel(input_ids, speaker_ids, charactor_embeddings, position_table):
    raise NotImplementedError("write your pallas kernel here")


# speedup vs baseline: 4.6284x; 4.6284x over previous
import functools

import jax
import jax.numpy as jnp
from jax import lax
from jax.experimental import pallas as pl
from jax.experimental.pallas import tpu as pltpu
from jax.experimental.pallas import tpu_sc as plsc

D = 128
SEQ = 200
HALF = 100
LANES = 16


@functools.lru_cache(maxsize=None)
def _make_emb_kernel(batch: int):
    info = plsc.get_sparse_core_info()
    nc, ns = info.num_cores, info.num_subcores
    nw = nc * ns
    seq_per_w = batch // nw

    mesh = plsc.VectorSubcoreMesh(core_axis_name="c", subcore_axis_name="s")

    @functools.partial(
        pl.kernel,
        out_type=jax.ShapeDtypeStruct((batch * SEQ, D), jnp.float32),
        mesh=mesh,
        scratch_types=[
            pltpu.VMEM((seq_per_w, 2, HALF), jnp.int32),
            pltpu.VMEM((SEQ, D), jnp.float32),
            pltpu.VMEM((SEQ, D), jnp.float32),
            pltpu.SemaphoreType.DMA,
        ],
    )
    def emb(ids_hbm, pos_hbm, table_hbm, out_hbm, idx_v, pos_v, rows_v, sem):
        wid = lax.axis_index("s") * nc + lax.axis_index("c")
        pltpu.sync_copy(ids_hbm.at[pl.ds(wid * seq_per_w, seq_per_w)], idx_v)
        pltpu.sync_copy(pos_hbm, pos_v)

        def chunk_body(c, carry):
            g0 = pltpu.async_copy(
                table_hbm.at[idx_v.at[c, 0]], rows_v.at[pl.ds(0, HALF)], sem)
            g1 = pltpu.async_copy(
                table_hbm.at[idx_v.at[c, 1]], rows_v.at[pl.ds(HALF, HALF)], sem)
            g0.wait()
            g1.wait()

            def row_body(r, rc):
                for j in range(D // LANES):
                    sl = pl.ds(j * LANES, LANES)
                    rows_v[r, sl] = rows_v[r, sl] + pos_v[r, sl]
                return rc

            lax.fori_loop(0, SEQ, row_body, 0)
            out_base = (wid * seq_per_w + c) * SEQ
            pltpu.sync_copy(rows_v, out_hbm.at[pl.ds(out_base, SEQ)])
            return carry

        lax.fori_loop(0, seq_per_w, chunk_body, 0)

    return emb


def kernel(input_ids, speaker_ids, charactor_embeddings, position_table):
    del speaker_ids
    batch, seq = input_ids.shape
    _, d = charactor_embeddings.shape
    ids3d = input_ids.reshape(batch, 2, HALF)
    pos = lax.slice(position_table, (1, 0), (1 + seq, d))
    out = _make_emb_kernel(batch)(ids3d, pos, charactor_embeddings)
    return out.reshape(batch, seq, d)


# speedup vs baseline: 5.6816x; 1.2275x over previous
import functools

import jax
import jax.numpy as jnp
from jax import lax
from jax.experimental import pallas as pl
from jax.experimental.pallas import tpu as pltpu
from jax.experimental.pallas import tpu_sc as plsc

D = 128
SEQ = 200
HALF = 100
LANES = 16


@functools.lru_cache(maxsize=None)
def _make_emb_kernel(batch: int):
    info = plsc.get_sparse_core_info()
    nc, ns = info.num_cores, info.num_subcores
    nw = nc * ns
    seq_per_w = batch // nw

    mesh = plsc.VectorSubcoreMesh(core_axis_name="c", subcore_axis_name="s")

    @functools.partial(
        pl.kernel,
        out_type=jax.ShapeDtypeStruct((batch * SEQ, D), jnp.float32),
        mesh=mesh,
        scratch_types=[
            pltpu.VMEM((seq_per_w, 2, HALF), jnp.int32),
            pltpu.VMEM((SEQ, D), jnp.float32),
            pltpu.VMEM((SEQ, D), jnp.float32),
            pltpu.SemaphoreType.DMA,
        ],
    )
    def emb(ids_hbm, pos_hbm, table_hbm, out_hbm, idx_v, pos_v, rows_v, sem):
        wid = lax.axis_index("s") * nc + lax.axis_index("c")
        pltpu.sync_copy(ids_hbm.at[pl.ds(wid * seq_per_w, seq_per_w)], idx_v)
        pltpu.sync_copy(pos_hbm, pos_v)

        def chunk_body(c, carry):
            def row_body(r, rc):
                for j in range(D // LANES):
                    sl = pl.ds(j * LANES, LANES)
                    rows_v[r, sl] = pos_v[r, sl]
                return rc

            lax.fori_loop(0, SEQ, row_body, 0)
            g0 = pltpu.async_copy(
                table_hbm.at[idx_v.at[c, 0]], rows_v.at[pl.ds(0, HALF)], sem,
                add=True)
            g1 = pltpu.async_copy(
                table_hbm.at[idx_v.at[c, 1]], rows_v.at[pl.ds(HALF, HALF)], sem,
                add=True)
            g0.wait()
            g1.wait()
            out_base = (wid * seq_per_w + c) * SEQ
            pltpu.sync_copy(rows_v, out_hbm.at[pl.ds(out_base, SEQ)])
            return carry

        lax.fori_loop(0, seq_per_w, chunk_body, 0)

    return emb


def kernel(input_ids, speaker_ids, charactor_embeddings, position_table):
    del speaker_ids
    batch, seq = input_ids.shape
    _, d = charactor_embeddings.shape
    ids3d = input_ids.reshape(batch, 2, HALF)
    pos = lax.slice(position_table, (1, 0), (1 + seq, d))
    out = _make_emb_kernel(batch)(ids3d, pos, charactor_embeddings)
    return out.reshape(batch, seq, d)


# speedup vs baseline: 9.2186x; 1.6225x over previous
import functools

import jax
import jax.numpy as jnp
from jax import lax
from jax.experimental import pallas as pl
from jax.experimental.pallas import tpu as pltpu
from jax.experimental.pallas import tpu_sc as plsc

D = 128
SEQ = 200
HALFSEQ = 100
LANES = 16


@functools.lru_cache(maxsize=None)
def _make_emb_kernel(batch: int):
    info = plsc.get_sparse_core_info()
    nc, ns = info.num_cores, info.num_subcores
    nw = nc * ns
    seq_per_w = batch // nw
    rows_per_w = seq_per_w * SEQ
    n_u = seq_per_w // 2

    mesh = plsc.VectorSubcoreMesh(core_axis_name="c", subcore_axis_name="s")

    @functools.partial(
        pl.kernel,
        out_type=jax.ShapeDtypeStruct((batch * SEQ, D), jnp.float32),
        mesh=mesh,
        scratch_types=[
            pltpu.VMEM((seq_per_w, 2, HALFSEQ), jnp.int32),
            pltpu.VMEM((SEQ, D), jnp.float32),
            pltpu.VMEM((2, SEQ, D), jnp.float32),
            pltpu.SemaphoreType.DMA,
            pltpu.SemaphoreType.DMA,
            pltpu.SemaphoreType.DMA,
            pltpu.SemaphoreType.DMA,
        ],
    )
    def emb(ids_hbm, pos_hbm, table_hbm, out_hbm, idx_v, pos_v, bufs,
            g0, g1, s0, s1):
        g = (g0, g1)
        st = (s0, s1)
        wid = lax.axis_index("s") * nc + lax.axis_index("c")
        pltpu.sync_copy(ids_hbm.at[pl.ds(wid * seq_per_w, seq_per_w)], idx_v)
        pltpu.sync_copy(pos_hbm, pos_v)
        out_w = wid * rows_per_w

        def prefill(half, _unused=None):
            def row_body(r, rc):
                for k in range(D // LANES):
                    sl = pl.ds(k * LANES, LANES)
                    bufs[half, r, sl] = pos_v[r, sl]
                return rc
            lax.fori_loop(0, SEQ, row_body, 0)

        def issue_gathers(half, s):
            for j in range(2):
                pltpu.async_copy(
                    table_hbm.at[idx_v.at[s, j]],
                    bufs.at[half, pl.ds(j * HALFSEQ, HALFSEQ)],
                    g[half], add=True)

        def wait_gathers(half):
            for j in range(2):
                pltpu.make_async_copy(
                    table_hbm.at[idx_v.at[0, 0]],
                    bufs.at[half, pl.ds(j * HALFSEQ, HALFSEQ)],
                    g[half]).wait()

        def issue_store(half, s):
            pltpu.async_copy(
                bufs.at[half], out_hbm.at[pl.ds(out_w + s * SEQ, SEQ)],
                st[half])

        def wait_store(half):
            pltpu.make_async_copy(
                bufs.at[half], out_hbm.at[pl.ds(0, SEQ)], st[half]).wait()

        prefill(0)
        issue_gathers(0, 0)

        def u_body(u, carry):
            for h in (0, 1):
                other = 1 - h
                s = 2 * u + h
                if h == 0:
                    @pl.when(u > 0)
                    def _():
                        wait_store(other)
                else:
                    wait_store(other)
                def prep():
                    prefill(other)
                    issue_gathers(other, s + 1)
                if h == 0:
                    prep()
                else:
                    @pl.when(u < n_u - 1)
                    def _():
                        prep()
                wait_gathers(h)
                issue_store(h, s)
            return carry

        lax.fori_loop(0, n_u, u_body, 0)
        wait_store(1)

    return emb


def kernel(input_ids, speaker_ids, charactor_embeddings, position_table):
    del speaker_ids
    batch, seq = input_ids.shape
    _, d = charactor_embeddings.shape
    ids3d = input_ids.reshape(batch, 2, HALFSEQ)
    pos = lax.slice(position_table, (1, 0), (1 + seq, d))
    out = _make_emb_kernel(batch)(ids3d, pos, charactor_embeddings)
    return out.reshape(batch, seq, d)
